# Initial kernel scaffold; baseline (speedup 1.0000x reference)
import jax
import jax.numpy as jnp
from jax.experimental import pallas as pl


def kernel(x, edge_index, W1, b1, W2, b2):
    raise NotImplementedError("write your pallas kernel here")


# speedup vs baseline: 8.9186x; 8.9186x over previous
import functools

import jax
import jax.numpy as jnp
from jax import lax
from jax.experimental import pallas as pl
from jax.experimental.pallas import tpu as pltpu
from jax.experimental.pallas import tpu_sc as plsc

N = 10000
E = 320000
D = 128

NC = 2
NS = 16
NW = NC * NS
CHUNK = 128
CPT = 80
EP = NW * CPT * CHUNK
NPAD = NW * 320
RPT = NPAD // NS

_mesh = plsc.VectorSubcoreMesh(
    core_axis_name="c", subcore_axis_name="s", num_cores=NC, num_subcores=NS)


@functools.partial(
    pl.kernel,
    out_type=jax.ShapeDtypeStruct((NC, NPAD, 16), jnp.float32),
    mesh=_mesh,
    scratch_types=[
        pltpu.VMEM((CPT, CHUNK), jnp.int32),
        pltpu.VMEM((CHUNK, 16), jnp.float32),
        pltpu.VMEM_SHARED((NPAD, 16), jnp.float32),
    ],
)
def _sc_degree(dst_hbm, z16_hbm, ones_hbm, degp_hbm, dst_v, obuf, dacc):
    c = lax.axis_index("c")
    s = lax.axis_index("s")
    wid = s * NC + c
    pltpu.sync_copy(dst_hbm.at[wid], dst_v)
    pltpu.sync_copy(ones_hbm, obuf)
    pltpu.sync_copy(z16_hbm, dacc.at[pl.ds(s * RPT, RPT)])
    plsc.subcore_barrier()

    def body(j, carry):
        pltpu.sync_copy(obuf, dacc.at[dst_v.at[j]], add=True)
        return carry

    lax.fori_loop(0, CPT, body, 0)
    plsc.subcore_barrier()
    pltpu.sync_copy(dacc.at[pl.ds(s * RPT, RPT)],
                    degp_hbm.at[c, pl.ds(s * RPT, RPT)])


@functools.partial(
    pl.kernel,
    out_type=jax.ShapeDtypeStruct((NC, NPAD, D), jnp.float32),
    mesh=_mesh,
    scratch_types=[
        pltpu.VMEM((CPT, CHUNK), jnp.int32),
        pltpu.VMEM((CPT, CHUNK), jnp.int32),
        pltpu.VMEM((CHUNK, D), jnp.float32),
        pltpu.VMEM_SHARED((NPAD, D), jnp.float32),
        pltpu.SemaphoreType.DMA,
    ],
)
def _sc_aggregate(g_hbm, src_hbm, dst_hbm, zrows_hbm, sp_hbm,
                  src_v, dst_v, buf, acc, sem):
    c = lax.axis_index("c")
    s = lax.axis_index("s")
    wid = s * NC + c
    pltpu.sync_copy(src_hbm.at[wid], src_v)
    pltpu.sync_copy(dst_hbm.at[wid], dst_v)
    pltpu.sync_copy(zrows_hbm, acc.at[pl.ds(s * RPT, RPT)])
    plsc.subcore_barrier()

    def body(j, carry):
        pltpu.async_copy(g_hbm.at[src_v.at[j]], buf, sem).wait()
        pltpu.sync_copy(buf, acc.at[dst_v.at[j]], add=True)
        return carry

    lax.fori_loop(0, CPT, body, 0)
    plsc.subcore_barrier()
    pltpu.sync_copy(acc.at[pl.ds(s * RPT, RPT)],
                    sp_hbm.at[c, pl.ds(s * RPT, RPT)])


def _dinv_col(degp):
    deg = degp[0, :N, 0:1] + degp[1, :N, 0:1] + 1.0
    return lax.rsqrt(deg)


def _tc_scale_matmul_body(degp_ref, x_ref, w_ref, g_ref):
    dinv = _dinv_col(degp_ref[...])
    u = x_ref[...] * dinv
    g_ref[...] = jnp.dot(u, w_ref[...], preferred_element_type=jnp.float32)


_tc_scale_matmul = pl.pallas_call(
    _tc_scale_matmul_body,
    out_shape=jax.ShapeDtypeStruct((N, D), jnp.float32),
)


def _tc_mid_body(degp_ref, sp_ref, g_ref, b_ref, w_ref, g2_ref):
    dinv = _dinv_col(degp_ref[...])
    sp = sp_ref[...]
    h = dinv * (sp[0, :N] + sp[1, :N] + g_ref[...]) + b_ref[...]
    u2 = dinv * jnp.maximum(h, 0.0)
    g2_ref[...] = jnp.dot(u2, w_ref[...], preferred_element_type=jnp.float32)


_tc_mid = pl.pallas_call(
    _tc_mid_body,
    out_shape=jax.ShapeDtypeStruct((N, D), jnp.float32),
)


def _tc_final_body(degp_ref, sp_ref, g_ref, b_ref, out_ref):
    dinv = _dinv_col(degp_ref[...])
    sp = sp_ref[...]
    out_ref[...] = dinv * (sp[0, :N] + sp[1, :N] + g_ref[...]) + b_ref[...]


_tc_final = pl.pallas_call(
    _tc_final_body,
    out_shape=jax.ShapeDtypeStruct((N, D), jnp.float32),
)


def kernel(x, edge_index, W1, b1, W2, b2):
    ei = edge_index.astype(jnp.int32)
    pad = EP - E
    src = jnp.concatenate([ei[0], jnp.zeros((pad,), jnp.int32)])
    dst = jnp.concatenate([ei[1], jnp.full((pad,), N, jnp.int32)])
    src = src.reshape(NW, CPT, CHUNK)
    dst = dst.reshape(NW, CPT, CHUNK)

    zrows = jnp.zeros((RPT, D), jnp.float32)
    z16 = jnp.zeros((RPT, 16), jnp.float32)
    ones16 = jnp.ones((CHUNK, 16), jnp.float32)

    degp = _sc_degree(dst, z16, ones16)

    g1 = _tc_scale_matmul(degp, x, W1)
    s1p = _sc_aggregate(g1, src, dst, zrows)
    g2 = _tc_mid(degp, s1p, g1, b1, W2)
    s2p = _sc_aggregate(g2, src, dst, zrows)
    return _tc_final(degp, s2p, g2, b2)
